# baseline (speedup 1.0000x reference)
import functools

import jax
import jax.numpy as jnp
from jax import lax
from jax.experimental import pallas as pl
from jax.experimental.pallas import tpu as pltpu
from jax.experimental.pallas import tpu_sc as plsc

_STATE = 128
_N_IO = 320000
_N_VAL = 160000

_V_SC = 107520
_V_TC = _N_VAL - _V_SC

_IO_STEPS = 40
_IO_BLK = _N_IO // _IO_STEPS


def _io_sum_kernel(io_ref, out_ref, acc):
    i = pl.program_id(0)

    @pl.when(i == 0)
    def _init():
        acc[...] = jnp.zeros_like(acc)

    acc[...] += jnp.sum(io_ref[...], axis=0, keepdims=True)

    @pl.when(i == _IO_STEPS - 1)
    def _finish():
        out_ref[...] = acc[...]


def _io_sum(io_embed):
    return pl.pallas_call(
        _io_sum_kernel,
        grid=(_IO_STEPS,),
        in_specs=[pl.BlockSpec((_IO_BLK, 2 * _STATE), lambda i: (i, 0))],
        out_specs=pl.BlockSpec((1, 2 * _STATE), lambda i: (0, 0)),
        out_shape=jax.ShapeDtypeStruct((1, 2 * _STATE), jnp.float32),
        scratch_shapes=[pltpu.VMEM((1, 2 * _STATE), jnp.float32)],
    )(io_embed)


_VT_STEPS = 41
_VT_BLK = _V_TC // _VT_STEPS
_VT_OFF = _V_SC // _VT_BLK


def _vtail_sum_kernel(vt_ref, out_ref, acc):
    i = pl.program_id(0)

    @pl.when(i == 0)
    def _init():
        acc[...] = jnp.zeros_like(acc)

    acc[...] += jnp.sum(vt_ref[...], axis=0, keepdims=True)

    @pl.when(i == _VT_STEPS - 1)
    def _finish():
        out_ref[...] = acc[...]


def _vtail_sum(value_embed):
    return pl.pallas_call(
        _vtail_sum_kernel,
        grid=(_VT_STEPS,),
        in_specs=[pl.BlockSpec((_VT_BLK, _STATE), lambda i: (i + _VT_OFF, 0))],
        out_specs=pl.BlockSpec((1, _STATE), lambda i: (0, 0)),
        out_shape=jax.ShapeDtypeStruct((1, _STATE), jnp.float32),
        scratch_shapes=[pltpu.VMEM((1, _STATE), jnp.float32)],
    )(value_embed)


_NS = 16
_VAL_PER_W = _V_SC // _NS
_VAL_CHUNK = 480
_VAL_NCHUNK = _VAL_PER_W // _VAL_CHUNK
_ROW_UNROLL = 4


def _sc_val_kernel(val_hbm, out_hbm, buf0, buf1, accv, sem0, sem1):
    wid = lax.axis_index("s")
    base = wid * _VAL_PER_W
    bufs = (buf0, buf1)
    sems = (sem0, sem1)
    last = base + (_VAL_NCHUNK - 1) * _VAL_CHUNK

    def start(chunk_idx, k):
        off = jnp.minimum(base + chunk_idx * _VAL_CHUNK, last)
        return pltpu.async_copy(
            val_hbm.at[pl.ds(off, _VAL_CHUNK)], bufs[k], sems[k])

    start(0, 0)
    start(1, 1)

    def accum(buf, acc):
        def body(r, acc):
            accs = list(acc)
            for u in range(_ROW_UNROLL):
                row = r * _ROW_UNROLL + u
                for j in range(8):
                    accs[j] = accs[j] + buf[row, pl.ds(16 * j, 16)]
            return tuple(accs)

        return lax.fori_loop(0, _VAL_CHUNK // _ROW_UNROLL, body, acc)

    def pair_body(i, acc):
        pltpu.make_async_copy(
            val_hbm.at[pl.ds(base, _VAL_CHUNK)], bufs[0], sems[0]).wait()
        acc = accum(bufs[0], acc)
        start(2 * i + 2, 0)
        pltpu.make_async_copy(
            val_hbm.at[pl.ds(base, _VAL_CHUNK)], bufs[1], sems[1]).wait()
        acc = accum(bufs[1], acc)
        start(2 * i + 3, 1)
        return acc

    acc = tuple(jnp.zeros((16,), jnp.float32) for _ in range(8))
    acc = lax.fori_loop(0, _VAL_NCHUNK // 2, pair_body, acc)

    pltpu.make_async_copy(
        val_hbm.at[pl.ds(base, _VAL_CHUNK)], bufs[0], sems[0]).wait()
    pltpu.make_async_copy(
        val_hbm.at[pl.ds(base, _VAL_CHUNK)], bufs[1], sems[1]).wait()

    for j in range(8):
        accv[pl.ds(16 * j, 16)] = acc[j]
    pltpu.sync_copy(accv, out_hbm.at[wid])


def _val_partials(value_embed):
    mesh = plsc.VectorSubcoreMesh(
        core_axis_name="c", subcore_axis_name="s", num_cores=1)
    run = functools.partial(
        pl.kernel,
        mesh=mesh,
        out_type=jax.ShapeDtypeStruct((_NS, _STATE), jnp.float32),
        scratch_types=[
            pltpu.VMEM((_VAL_CHUNK, _STATE), jnp.float32),
            pltpu.VMEM((_VAL_CHUNK, _STATE), jnp.float32),
            pltpu.VMEM((_STATE,), jnp.float32),
            pltpu.SemaphoreType.DMA,
            pltpu.SemaphoreType.DMA,
        ],
    )(_sc_val_kernel)
    return run(value_embed)


def _combine_kernel(io_sum_ref, vt_sum_ref, part_ref, w_ref, b_ref, out_ref):
    io_mean = io_sum_ref[...] / _N_IO
    val_sum = vt_sum_ref[...] + jnp.sum(part_ref[...], axis=0,
                                        keepdims=True)
    val_mean = val_sum / _N_VAL
    joint = jnp.concatenate([io_mean, val_mean], axis=1)
    out_ref[...] = (
        lax.dot_general(joint, w_ref[...], (((1,), (1,)), ((), ())),
                        preferred_element_type=jnp.float32)
        + b_ref[...]
    )


def _combine(io_sum, vt_sum, parts, W, b2):
    return pl.pallas_call(
        _combine_kernel,
        out_shape=jax.ShapeDtypeStruct((1, _STATE), jnp.float32),
    )(io_sum, vt_sum, parts, W, b2)


def kernel(io_embed, value_embed, W, b):
    parts = _val_partials(value_embed)
    io_sum = _io_sum(io_embed)
    vt_sum = _vtail_sum(value_embed)
    return _combine(io_sum, vt_sum, parts, W, b.reshape(1, _STATE))


# speedup vs baseline: 1.1010x; 1.1010x over previous
import functools

import jax
import jax.numpy as jnp
from jax import lax
from jax.experimental import pallas as pl
from jax.experimental.pallas import tpu as pltpu
from jax.experimental.pallas import tpu_sc as plsc

_STATE = 128
_N_IO = 320000
_N_VAL = 160000

_V_SC = _N_VAL

_IO_STEPS = 40
_IO_BLK = _N_IO // _IO_STEPS


def _io_sum_kernel(io_ref, out_ref, acc):
    i = pl.program_id(0)

    @pl.when(i == 0)
    def _init():
        acc[...] = jnp.zeros_like(acc)

    acc[...] += jnp.sum(io_ref[...], axis=0, keepdims=True)

    @pl.when(i == _IO_STEPS - 1)
    def _finish():
        out_ref[...] = acc[...]


def _io_sum(io_embed):
    return pl.pallas_call(
        _io_sum_kernel,
        grid=(_IO_STEPS,),
        in_specs=[pl.BlockSpec((_IO_BLK, 2 * _STATE), lambda i: (i, 0))],
        out_specs=pl.BlockSpec((1, 2 * _STATE), lambda i: (0, 0)),
        out_shape=jax.ShapeDtypeStruct((1, 2 * _STATE), jnp.float32),
        scratch_shapes=[pltpu.VMEM((1, 2 * _STATE), jnp.float32)],
    )(io_embed)


_NS = 16
_VAL_PER_W = _V_SC // _NS
_VAL_CHUNK = 400
_VAL_NCHUNK = _VAL_PER_W // _VAL_CHUNK
_ROW_UNROLL = 4


def _sc_val_kernel(val_hbm, out_hbm, buf0, buf1, accv, sem0, sem1):
    wid = lax.axis_index("s")
    base = wid * _VAL_PER_W
    bufs = (buf0, buf1)
    sems = (sem0, sem1)
    last = base + (_VAL_NCHUNK - 1) * _VAL_CHUNK

    def start(chunk_idx, k):
        off = jnp.minimum(base + chunk_idx * _VAL_CHUNK, last)
        return pltpu.async_copy(
            val_hbm.at[pl.ds(off, _VAL_CHUNK)], bufs[k], sems[k])

    start(0, 0)
    start(1, 1)

    def accum(buf, acc):
        def body(r, acc):
            accs = list(acc)
            for u in range(_ROW_UNROLL):
                row = r * _ROW_UNROLL + u
                for j in range(8):
                    accs[j] = accs[j] + buf[row, pl.ds(16 * j, 16)]
            return tuple(accs)

        return lax.fori_loop(0, _VAL_CHUNK // _ROW_UNROLL, body, acc)

    def pair_body(i, acc):
        pltpu.make_async_copy(
            val_hbm.at[pl.ds(base, _VAL_CHUNK)], bufs[0], sems[0]).wait()
        acc = accum(bufs[0], acc)
        start(2 * i + 2, 0)
        pltpu.make_async_copy(
            val_hbm.at[pl.ds(base, _VAL_CHUNK)], bufs[1], sems[1]).wait()
        acc = accum(bufs[1], acc)
        start(2 * i + 3, 1)
        return acc

    acc = tuple(jnp.zeros((16,), jnp.float32) for _ in range(8))
    acc = lax.fori_loop(0, _VAL_NCHUNK // 2, pair_body, acc)

    pltpu.make_async_copy(
        val_hbm.at[pl.ds(base, _VAL_CHUNK)], bufs[0], sems[0]).wait()
    acc = accum(bufs[0], acc)
    pltpu.make_async_copy(
        val_hbm.at[pl.ds(base, _VAL_CHUNK)], bufs[1], sems[1]).wait()

    for j in range(8):
        accv[pl.ds(16 * j, 16)] = acc[j]
    pltpu.sync_copy(accv, out_hbm.at[wid])


def _val_partials(value_embed):
    mesh = plsc.VectorSubcoreMesh(
        core_axis_name="c", subcore_axis_name="s", num_cores=1)
    run = functools.partial(
        pl.kernel,
        mesh=mesh,
        out_type=jax.ShapeDtypeStruct((_NS, _STATE), jnp.float32),
        scratch_types=[
            pltpu.VMEM((_VAL_CHUNK, _STATE), jnp.float32),
            pltpu.VMEM((_VAL_CHUNK, _STATE), jnp.float32),
            pltpu.VMEM((_STATE,), jnp.float32),
            pltpu.SemaphoreType.DMA,
            pltpu.SemaphoreType.DMA,
        ],
    )(_sc_val_kernel)
    return run(value_embed)


def _combine_kernel(io_sum_ref, part_ref, w_ref, b_ref, out_ref):
    io_mean = io_sum_ref[...] / _N_IO
    val_sum = jnp.sum(part_ref[...], axis=0, keepdims=True)
    val_mean = val_sum / _N_VAL
    joint = jnp.concatenate([io_mean, val_mean], axis=1)
    out_ref[...] = (
        lax.dot_general(joint, w_ref[...], (((1,), (1,)), ((), ())),
                        preferred_element_type=jnp.float32)
        + b_ref[...]
    )


def _combine(io_sum, parts, W, b2):
    return pl.pallas_call(
        _combine_kernel,
        out_shape=jax.ShapeDtypeStruct((1, _STATE), jnp.float32),
    )(io_sum, parts, W, b2)


def kernel(io_embed, value_embed, W, b):
    parts = _val_partials(value_embed)
    io_sum = _io_sum(io_embed)
    return _combine(io_sum, parts, W, b.reshape(1, _STATE))


# speedup vs baseline: 1.1132x; 1.0111x over previous
import functools

import jax
import jax.numpy as jnp
from jax import lax
from jax.experimental import pallas as pl
from jax.experimental.pallas import tpu as pltpu
from jax.experimental.pallas import tpu_sc as plsc

_STATE = 128
_N_IO = 320000
_N_VAL = 160000

_V_SC = 32000
_V_TC = _N_VAL - _V_SC

_TC_STEPS = 40
_IO_BLK = _N_IO // _TC_STEPS
_VT_BLK = _V_TC // _TC_STEPS
_VT_OFF = _V_SC // _VT_BLK


def _tc_sum_kernel(io_ref, vt_ref, io_out, vt_out, io_acc, vt_acc):
    i = pl.program_id(0)

    @pl.when(i == 0)
    def _init():
        io_acc[...] = jnp.zeros_like(io_acc)
        vt_acc[...] = jnp.zeros_like(vt_acc)

    io_acc[...] += jnp.sum(io_ref[...], axis=0, keepdims=True)
    vt_acc[...] += jnp.sum(vt_ref[...], axis=0, keepdims=True)

    @pl.when(i == _TC_STEPS - 1)
    def _finish():
        io_out[...] = io_acc[...]
        vt_out[...] = vt_acc[...]


def _tc_sums(io_embed, value_embed):
    return pl.pallas_call(
        _tc_sum_kernel,
        grid=(_TC_STEPS,),
        in_specs=[
            pl.BlockSpec((_IO_BLK, 2 * _STATE), lambda i: (i, 0)),
            pl.BlockSpec((_VT_BLK, _STATE), lambda i: (i + _VT_OFF, 0)),
        ],
        out_specs=[
            pl.BlockSpec((1, 2 * _STATE), lambda i: (0, 0)),
            pl.BlockSpec((1, _STATE), lambda i: (0, 0)),
        ],
        out_shape=[
            jax.ShapeDtypeStruct((1, 2 * _STATE), jnp.float32),
            jax.ShapeDtypeStruct((1, _STATE), jnp.float32),
        ],
        scratch_shapes=[
            pltpu.VMEM((1, 2 * _STATE), jnp.float32),
            pltpu.VMEM((1, _STATE), jnp.float32),
        ],
    )(io_embed, value_embed)


_NS = 16
_VAL_PER_W = _V_SC // _NS
_VAL_CHUNK = 400
_VAL_NCHUNK = _VAL_PER_W // _VAL_CHUNK
_ROW_UNROLL = 4


def _sc_val_kernel(val_hbm, out_hbm, buf0, buf1, accv, sem0, sem1):
    wid = lax.axis_index("s")
    base = wid * _VAL_PER_W
    bufs = (buf0, buf1)
    sems = (sem0, sem1)
    last = base + (_VAL_NCHUNK - 1) * _VAL_CHUNK

    def start(chunk_idx, k):
        off = jnp.minimum(base + chunk_idx * _VAL_CHUNK, last)
        return pltpu.async_copy(
            val_hbm.at[pl.ds(off, _VAL_CHUNK)], bufs[k], sems[k])

    start(0, 0)
    start(1, 1)

    def accum(buf, acc):
        def body(r, acc):
            accs = list(acc)
            for u in range(_ROW_UNROLL):
                row = r * _ROW_UNROLL + u
                for j in range(8):
                    accs[j] = accs[j] + buf[row, pl.ds(16 * j, 16)]
            return tuple(accs)

        return lax.fori_loop(0, _VAL_CHUNK // _ROW_UNROLL, body, acc)

    def pair_body(i, acc):
        pltpu.make_async_copy(
            val_hbm.at[pl.ds(base, _VAL_CHUNK)], bufs[0], sems[0]).wait()
        acc = accum(bufs[0], acc)
        start(2 * i + 2, 0)
        pltpu.make_async_copy(
            val_hbm.at[pl.ds(base, _VAL_CHUNK)], bufs[1], sems[1]).wait()
        acc = accum(bufs[1], acc)
        start(2 * i + 3, 1)
        return acc

    acc = tuple(jnp.zeros((16,), jnp.float32) for _ in range(8))
    acc = lax.fori_loop(0, _VAL_NCHUNK // 2, pair_body, acc)

    pltpu.make_async_copy(
        val_hbm.at[pl.ds(base, _VAL_CHUNK)], bufs[0], sems[0]).wait()
    acc = accum(bufs[0], acc)
    pltpu.make_async_copy(
        val_hbm.at[pl.ds(base, _VAL_CHUNK)], bufs[1], sems[1]).wait()

    for j in range(8):
        accv[pl.ds(16 * j, 16)] = acc[j]
    pltpu.sync_copy(accv, out_hbm.at[wid])


def _val_partials(value_embed):
    mesh = plsc.VectorSubcoreMesh(
        core_axis_name="c", subcore_axis_name="s", num_cores=1)
    run = functools.partial(
        pl.kernel,
        mesh=mesh,
        out_type=jax.ShapeDtypeStruct((_NS, _STATE), jnp.float32),
        scratch_types=[
            pltpu.VMEM((_VAL_CHUNK, _STATE), jnp.float32),
            pltpu.VMEM((_VAL_CHUNK, _STATE), jnp.float32),
            pltpu.VMEM((_STATE,), jnp.float32),
            pltpu.SemaphoreType.DMA,
            pltpu.SemaphoreType.DMA,
        ],
    )(_sc_val_kernel)
    return run(value_embed)


def _combine_kernel(io_sum_ref, vt_sum_ref, part_ref, w_ref, b_ref, out_ref):
    io_mean = io_sum_ref[...] / _N_IO
    val_sum = vt_sum_ref[...] + jnp.sum(part_ref[...], axis=0,
                                        keepdims=True)
    val_mean = val_sum / _N_VAL
    joint = jnp.concatenate([io_mean, val_mean], axis=1)
    out_ref[...] = (
        lax.dot_general(joint, w_ref[...], (((1,), (1,)), ((), ())),
                        preferred_element_type=jnp.float32)
        + b_ref[...]
    )


def _combine(io_sum, vt_sum, parts, W, b2):
    return pl.pallas_call(
        _combine_kernel,
        out_shape=jax.ShapeDtypeStruct((1, _STATE), jnp.float32),
    )(io_sum, vt_sum, parts, W, b2)


def kernel(io_embed, value_embed, W, b):
    parts = _val_partials(value_embed)
    io_sum, vt_sum = _tc_sums(io_embed, value_embed)
    return _combine(io_sum, vt_sum, parts, W, b.reshape(1, _STATE))
